# baseline (speedup 1.0000x reference)
import jax
import jax.numpy as jnp
from jax import lax
from jax.experimental import pallas as pl
from jax.experimental.pallas import tpu as pltpu
from jax.experimental.pallas import tpu_sc as plsc

N = 10000
E = 320000
D = 128

CW = 50
NCH = E // CW
NW = 32
CPW = NCH // NW
GRP = 25
CPG = CPW // GRP
NBUF = 4
RC = 128
ROW_FULL = N // RC
ROW_TAIL = N - ROW_FULL * RC


def _sc_agg_body(x_hbm, src_hbm, dst_hbm, zeros_hbm, out_hbm,
                 srcb, dstb, r0, r1, r2, r3, acc,
                 gsem0, gsem1, gsem2, gsem3, ssem0, ssem1, ssem2, ssem3):
    c = lax.axis_index("c")
    s = lax.axis_index("s")
    wid = s * 2 + c
    rows = [r0, r1, r2, r3]
    gsems = [gsem0, gsem1, gsem2, gsem3]
    ssems = [ssem0, ssem1, ssem2, ssem3]
    base = wid * CPW

    def zero_body(j, carry):
        r = (s + j * 16) * RC

        @pl.when(c == 0)
        def _():
            pltpu.sync_copy(x_hbm.at[pl.ds(r, RC)], acc.at[pl.ds(r, RC)])

        @pl.when(c == 1)
        def _():
            pltpu.sync_copy(zeros_hbm, acc.at[pl.ds(r, RC)])

        return carry

    nz = jnp.where(s < ROW_FULL - (ROW_FULL // 16) * 16, ROW_FULL // 16 + 1,
                   ROW_FULL // 16)
    lax.fori_loop(0, nz, zero_body, 0)

    @pl.when(s == 15)
    def _():
        tail = pl.ds(ROW_FULL * RC, ROW_TAIL)

        @pl.when(c == 0)
        def _():
            pltpu.sync_copy(x_hbm.at[tail], acc.at[tail])

        @pl.when(c == 1)
        def _():
            pltpu.sync_copy(zeros_hbm.at[pl.ds(0, ROW_TAIL)], acc.at[tail])

    plsc.subcore_barrier()

    def group(g, carry):
        gb = base + g * CPG
        pltpu.sync_copy(src_hbm.at[pl.ds(gb, CPG)], srcb)
        pltpu.sync_copy(dst_hbm.at[pl.ds(gb, CPG)], dstb)

        for b in range(NBUF):
            pltpu.async_copy(x_hbm.at[srcb.at[b]], rows[b], gsems[b])

        def window(i, carry2):
            k0 = i * NBUF
            for b in range(NBUF):
                k = k0 + b
                pltpu.make_async_copy(x_hbm.at[srcb.at[k]], rows[b],
                                      gsems[b]).wait()
                pltpu.async_copy(rows[b], acc.at[dstb.at[k]], ssems[b],
                                 add=True)
            for b in range(NBUF):
                k = k0 + b

                @pl.when(k + NBUF < CPG)
                def _():
                    pltpu.make_async_copy(rows[b], acc.at[dstb.at[k]],
                                          ssems[b]).wait()
                    pltpu.async_copy(x_hbm.at[srcb.at[k + NBUF]], rows[b],
                                     gsems[b])
            return carry2

        lax.fori_loop(0, CPG // NBUF, window, 0)

        for b in range(NBUF):
            k = CPG - NBUF + b
            pltpu.make_async_copy(rows[b], acc.at[dstb.at[k]],
                                  ssems[b]).wait()
        return carry

    lax.fori_loop(0, GRP, group, 0)

    plsc.subcore_barrier()

    def wb_body(j, carry):
        r = (s + j * 16) * RC
        pltpu.sync_copy(acc.at[pl.ds(r, RC)], out_hbm.at[c, pl.ds(r, RC)])
        return carry

    lax.fori_loop(0, nz, wb_body, 0)

    @pl.when(s == 15)
    def _():
        pltpu.sync_copy(acc.at[pl.ds(ROW_FULL * RC, ROW_TAIL)],
                        out_hbm.at[c, pl.ds(ROW_FULL * RC, ROW_TAIL)])


def _sc_agg(x, src2d, dst2d, zeros):
    k = pl.kernel(
        _sc_agg_body,
        out_type=jax.ShapeDtypeStruct((2, N, D), jnp.float32),
        mesh=plsc.VectorSubcoreMesh(core_axis_name="c", subcore_axis_name="s"),
        scratch_types=(
            [pltpu.VMEM((CPG, CW), jnp.int32),
             pltpu.VMEM((CPG, CW), jnp.int32)]
            + [pltpu.VMEM((CW, D), jnp.float32) for _ in range(NBUF)]
            + [pltpu.VMEM_SHARED((N, D), jnp.float32)]
            + [pltpu.SemaphoreType.DMA for _ in range(2 * NBUF)]
        ),
    )
    return k(x, src2d, dst2d, zeros)


BN = 2000


def _mlp1_body(p_ref, w1_ref, b1_ref, w2_ref, b2_ref, o_ref):
    t = p_ref[0] + p_ref[1]
    h = jnp.dot(t, w1_ref[...], preferred_element_type=jnp.float32) + b1_ref[...]
    h = jnp.maximum(h, 0.0)
    h = jnp.dot(h, w2_ref[...], preferred_element_type=jnp.float32) + b2_ref[...]
    o_ref[...] = jnp.maximum(h, 0.0)


def _tc_mlp1(parts, w1, b1, w2, b2):
    grid = (N // BN,)
    return pl.pallas_call(
        _mlp1_body,
        grid=grid,
        in_specs=[
            pl.BlockSpec((2, BN, D), lambda i: (0, i, 0)),
            pl.BlockSpec((D, D), lambda i: (0, 0)),
            pl.BlockSpec((1, D), lambda i: (0, 0)),
            pl.BlockSpec((D, D), lambda i: (0, 0)),
            pl.BlockSpec((1, D), lambda i: (0, 0)),
        ],
        out_specs=pl.BlockSpec((BN, D), lambda i: (i, 0)),
        out_shape=jax.ShapeDtypeStruct((N, D), jnp.float32),
    )(parts, w1, b1.reshape(1, D), w2, b2.reshape(1, D))


def _mlp2_body(p_ref, w1_ref, b1_ref, w2_ref, b2_ref,
               wh1_ref, bh1_ref, wh2_ref, bh2_ref, o_ref):
    t = p_ref[0] + p_ref[1]
    z = jnp.dot(t, w1_ref[...], preferred_element_type=jnp.float32) + b1_ref[...]
    z = jnp.maximum(z, 0.0)
    z = jnp.dot(z, w2_ref[...], preferred_element_type=jnp.float32) + b2_ref[...]
    z = jnp.maximum(z, 0.0)
    z = jnp.dot(z, wh1_ref[...], preferred_element_type=jnp.float32) + bh1_ref[...]
    z = jnp.maximum(z, 0.0)
    o_ref[...] = (jnp.dot(z, wh2_ref[...], preferred_element_type=jnp.float32)
                  + bh2_ref[...])


def _tc_mlp2(parts, w1, b1, w2, b2, wh1, bh1, wh2, bh2):
    grid = (N // BN,)
    return pl.pallas_call(
        _mlp2_body,
        grid=grid,
        in_specs=[
            pl.BlockSpec((2, BN, D), lambda i: (0, i, 0)),
            pl.BlockSpec((D, D), lambda i: (0, 0)),
            pl.BlockSpec((1, D), lambda i: (0, 0)),
            pl.BlockSpec((D, D), lambda i: (0, 0)),
            pl.BlockSpec((1, D), lambda i: (0, 0)),
            pl.BlockSpec((D, D), lambda i: (0, 0)),
            pl.BlockSpec((1, D), lambda i: (0, 0)),
            pl.BlockSpec((D, D), lambda i: (0, 0)),
            pl.BlockSpec((1, D), lambda i: (0, 0)),
        ],
        out_specs=pl.BlockSpec((BN, D), lambda i: (i, 0)),
        out_shape=jax.ShapeDtypeStruct((N, D), jnp.float32),
    )(parts, w1, b1.reshape(1, D), w2, b2.reshape(1, D),
      wh1, bh1.reshape(1, D), wh2, bh2.reshape(1, D))


def kernel(x, edge_index, W1a, b1a, W2a, b2a, W1b, b1b, W2b, b2b,
           Wh1, bh1, Wh2, bh2):
    src2d = edge_index[0].astype(jnp.int32).reshape(NCH, CW)
    dst2d = edge_index[1].astype(jnp.int32).reshape(NCH, CW)
    zeros = jnp.zeros((RC, D), jnp.float32)

    parts1 = _sc_agg(x, src2d, dst2d, zeros)
    h1 = _tc_mlp1(parts1, W1a, b1a, W2a, b2a)
    parts2 = _sc_agg(h1, src2d, dst2d, zeros)
    out = _tc_mlp2(parts2, W1b, b1b, W2b, b2b, Wh1, bh1, Wh2, bh2)
    return out


# speedup vs baseline: 1.1133x; 1.1133x over previous
import jax
import jax.numpy as jnp
from jax import lax
from jax.experimental import pallas as pl
from jax.experimental.pallas import tpu as pltpu
from jax.experimental.pallas import tpu_sc as plsc

N = 10000
E = 320000
D = 128

CW = 125
NCH = E // CW
NW = 32
CPW = NCH // NW
GRP = 2
CPG = CPW // GRP
NBUF = 2
RC = 128
ROW_FULL = N // RC
ROW_TAIL = N - ROW_FULL * RC


def _sc_agg_body(x_hbm, src_hbm, dst_hbm, zeros_hbm, out_hbm,
                 srcb, dstb, r0, r1, acc, gsem0, gsem1, ssem0, ssem1):
    c = lax.axis_index("c")
    s = lax.axis_index("s")
    wid = s * 2 + c
    rows = [r0, r1]
    gsems = [gsem0, gsem1]
    ssems = [ssem0, ssem1]
    base = wid * CPW

    def zero_body(j, carry):
        r = (s + j * 16) * RC

        @pl.when(c == 0)
        def _():
            pltpu.sync_copy(x_hbm.at[pl.ds(r, RC)], acc.at[pl.ds(r, RC)])

        @pl.when(c == 1)
        def _():
            pltpu.sync_copy(zeros_hbm, acc.at[pl.ds(r, RC)])

        return carry

    nz = jnp.where(s < ROW_FULL - (ROW_FULL // 16) * 16, ROW_FULL // 16 + 1,
                   ROW_FULL // 16)
    lax.fori_loop(0, nz, zero_body, 0)

    @pl.when(s == 15)
    def _():
        tail = pl.ds(ROW_FULL * RC, ROW_TAIL)

        @pl.when(c == 0)
        def _():
            pltpu.sync_copy(x_hbm.at[tail], acc.at[tail])

        @pl.when(c == 1)
        def _():
            pltpu.sync_copy(zeros_hbm.at[pl.ds(0, ROW_TAIL)], acc.at[tail])

    plsc.subcore_barrier()

    def group(g, carry):
        gb = base + g * CPG
        pltpu.sync_copy(src_hbm.at[pl.ds(gb, CPG)], srcb)
        pltpu.sync_copy(dst_hbm.at[pl.ds(gb, CPG)], dstb)

        for b in range(NBUF):
            pltpu.async_copy(x_hbm.at[srcb.at[b]], rows[b], gsems[b])

        def window(i, carry2):
            k0 = i * NBUF
            for b in range(NBUF):
                k = k0 + b
                pltpu.make_async_copy(x_hbm.at[srcb.at[k]], rows[b],
                                      gsems[b]).wait()
                pltpu.async_copy(rows[b], acc.at[dstb.at[k]], ssems[b],
                                 add=True)
            for b in range(NBUF):
                k = k0 + b

                @pl.when(k + NBUF < CPG)
                def _():
                    pltpu.make_async_copy(rows[b], acc.at[dstb.at[k]],
                                          ssems[b]).wait()
                    pltpu.async_copy(x_hbm.at[srcb.at[k + NBUF]], rows[b],
                                     gsems[b])
            return carry2

        lax.fori_loop(0, CPG // NBUF, window, 0)

        for b in range(NBUF):
            k = CPG - NBUF + b
            pltpu.make_async_copy(rows[b], acc.at[dstb.at[k]],
                                  ssems[b]).wait()
        return carry

    lax.fori_loop(0, GRP, group, 0)

    plsc.subcore_barrier()

    def wb_body(j, carry):
        r = (s + j * 16) * RC
        pltpu.sync_copy(acc.at[pl.ds(r, RC)], out_hbm.at[c, pl.ds(r, RC)])
        return carry

    lax.fori_loop(0, nz, wb_body, 0)

    @pl.when(s == 15)
    def _():
        pltpu.sync_copy(acc.at[pl.ds(ROW_FULL * RC, ROW_TAIL)],
                        out_hbm.at[c, pl.ds(ROW_FULL * RC, ROW_TAIL)])


def _sc_agg(x, src2d, dst2d, zeros):
    k = pl.kernel(
        _sc_agg_body,
        out_type=jax.ShapeDtypeStruct((2, N, D), jnp.float32),
        mesh=plsc.VectorSubcoreMesh(core_axis_name="c", subcore_axis_name="s"),
        scratch_types=(
            [pltpu.VMEM((CPG, CW), jnp.int32),
             pltpu.VMEM((CPG, CW), jnp.int32)]
            + [pltpu.VMEM((CW, D), jnp.float32) for _ in range(NBUF)]
            + [pltpu.VMEM_SHARED((N, D), jnp.float32)]
            + [pltpu.SemaphoreType.DMA for _ in range(2 * NBUF)]
        ),
    )
    return k(x, src2d, dst2d, zeros)


BN = 2000


def _mlp1_body(p_ref, w1_ref, b1_ref, w2_ref, b2_ref, o_ref):
    t = p_ref[0] + p_ref[1]
    h = jnp.dot(t, w1_ref[...], preferred_element_type=jnp.float32) + b1_ref[...]
    h = jnp.maximum(h, 0.0)
    h = jnp.dot(h, w2_ref[...], preferred_element_type=jnp.float32) + b2_ref[...]
    o_ref[...] = jnp.maximum(h, 0.0)


def _tc_mlp1(parts, w1, b1, w2, b2):
    grid = (N // BN,)
    return pl.pallas_call(
        _mlp1_body,
        grid=grid,
        in_specs=[
            pl.BlockSpec((2, BN, D), lambda i: (0, i, 0)),
            pl.BlockSpec((D, D), lambda i: (0, 0)),
            pl.BlockSpec((1, D), lambda i: (0, 0)),
            pl.BlockSpec((D, D), lambda i: (0, 0)),
            pl.BlockSpec((1, D), lambda i: (0, 0)),
        ],
        out_specs=pl.BlockSpec((BN, D), lambda i: (i, 0)),
        out_shape=jax.ShapeDtypeStruct((N, D), jnp.float32),
    )(parts, w1, b1.reshape(1, D), w2, b2.reshape(1, D))


def _mlp2_body(p_ref, w1_ref, b1_ref, w2_ref, b2_ref,
               wh1_ref, bh1_ref, wh2_ref, bh2_ref, o_ref):
    t = p_ref[0] + p_ref[1]
    z = jnp.dot(t, w1_ref[...], preferred_element_type=jnp.float32) + b1_ref[...]
    z = jnp.maximum(z, 0.0)
    z = jnp.dot(z, w2_ref[...], preferred_element_type=jnp.float32) + b2_ref[...]
    z = jnp.maximum(z, 0.0)
    z = jnp.dot(z, wh1_ref[...], preferred_element_type=jnp.float32) + bh1_ref[...]
    z = jnp.maximum(z, 0.0)
    o_ref[...] = (jnp.dot(z, wh2_ref[...], preferred_element_type=jnp.float32)
                  + bh2_ref[...])


def _tc_mlp2(parts, w1, b1, w2, b2, wh1, bh1, wh2, bh2):
    grid = (N // BN,)
    return pl.pallas_call(
        _mlp2_body,
        grid=grid,
        in_specs=[
            pl.BlockSpec((2, BN, D), lambda i: (0, i, 0)),
            pl.BlockSpec((D, D), lambda i: (0, 0)),
            pl.BlockSpec((1, D), lambda i: (0, 0)),
            pl.BlockSpec((D, D), lambda i: (0, 0)),
            pl.BlockSpec((1, D), lambda i: (0, 0)),
            pl.BlockSpec((D, D), lambda i: (0, 0)),
            pl.BlockSpec((1, D), lambda i: (0, 0)),
            pl.BlockSpec((D, D), lambda i: (0, 0)),
            pl.BlockSpec((1, D), lambda i: (0, 0)),
        ],
        out_specs=pl.BlockSpec((BN, D), lambda i: (i, 0)),
        out_shape=jax.ShapeDtypeStruct((N, D), jnp.float32),
    )(parts, w1, b1.reshape(1, D), w2, b2.reshape(1, D),
      wh1, bh1.reshape(1, D), wh2, bh2.reshape(1, D))


def kernel(x, edge_index, W1a, b1a, W2a, b2a, W1b, b1b, W2b, b2b,
           Wh1, bh1, Wh2, bh2):
    src2d = edge_index[0].astype(jnp.int32).reshape(NCH, CW)
    dst2d = edge_index[1].astype(jnp.int32).reshape(NCH, CW)
    zeros = jnp.zeros((RC, D), jnp.float32)

    parts1 = _sc_agg(x, src2d, dst2d, zeros)
    h1 = _tc_mlp1(parts1, W1a, b1a, W2a, b2a)
    parts2 = _sc_agg(h1, src2d, dst2d, zeros)
    out = _tc_mlp2(parts2, W1b, b1b, W2b, b2b, Wh1, bh1, Wh2, bh2)
    return out
